# baseline (speedup 1.0000x reference)
import jax
import jax.numpy as jnp
from jax.experimental import pallas as pl
from jax.experimental.pallas import tpu as pltpu

SOM_H, SOM_W, D = 64, 64, 512
HW = SOM_H * SOM_W
BATCH = 4096
TB = 1024


def _som_kernel(x_ref, w_ref, row_ref, col_ref, idx_ref, wsq_ref):
    @pl.when(pl.program_id(0) == 0)
    def _():
        w = w_ref[...]
        wsq_ref[...] = jnp.sum(w * w, axis=1)[None, :]

    x = x_ref[...]
    xn = -2.0 * x
    x_sq = jnp.sum(x * x, axis=1, keepdims=True)
    xnt = xn.T
    cross2 = jax.lax.dot_general(
        xnt, w_ref[...], (((0,), (1,)), ((), ())),
        preferred_element_type=jnp.float32,
    )
    dist = (x_sq + cross2) + wsq_ref[...]
    idx = jnp.argmin(dist, axis=1).astype(jnp.int32)
    idx_ref[...] = idx
    row_ref[...] = idx // SOM_W
    col_ref[...] = idx % SOM_W


def kernel(x, weights):
    wneg = weights.reshape(HW, D)
    grid = (BATCH // TB,)
    row, col, idx = pl.pallas_call(
        _som_kernel,
        grid=grid,
        in_specs=[
            pl.BlockSpec((TB, D), lambda i: (i, 0)),
            pl.BlockSpec((HW, D), lambda i: (0, 0)),
        ],
        out_specs=[
            pl.BlockSpec((TB,), lambda i: (i,)),
            pl.BlockSpec((TB,), lambda i: (i,)),
            pl.BlockSpec((TB,), lambda i: (i,)),
        ],
        out_shape=[
            jax.ShapeDtypeStruct((BATCH,), jnp.int32),
            jax.ShapeDtypeStruct((BATCH,), jnp.int32),
            jax.ShapeDtypeStruct((BATCH,), jnp.int32),
        ],
        scratch_shapes=[pltpu.VMEM((1, HW), jnp.float32)],
    )(x, wneg)
    return jnp.stack([row, col], axis=1), idx
